# baseline (speedup 1.0000x reference)
import functools

import jax
import jax.numpy as jnp
from jax import lax
from jax.experimental import pallas as pl
from jax.experimental.pallas import tpu as pltpu
from jax.experimental.pallas import tpu_sc as plsc

_LANES = 16
_NUM_CORES = 2
_NUM_SUBCORES = 16
_NUM_WORKERS = _NUM_CORES * _NUM_SUBCORES
_UNROLL = 20
_NBUF = 6
_TCHUNK = 16384


def _untile_table(emb_t):
    dim, vocab = emb_t.shape
    half = _TCHUNK // 2
    grid = (vocab + _TCHUNK - 1) // _TCHUNK

    def body(in_ref, out_ref):
        b = in_ref[...]
        stacked = jnp.concatenate(
            [b[:, :half], b[:, half:]], axis=0)
        out_ref[...] = jnp.swapaxes(stacked, 0, 1)

    out2 = pl.pallas_call(
        body,
        grid=(grid,),
        in_specs=[pl.BlockSpec((dim, _TCHUNK), lambda c: (0, c))],
        out_specs=pl.BlockSpec((half, 2 * dim), lambda c: (c, 0)),
        out_shape=jax.ShapeDtypeStruct((grid * half, 2 * dim), jnp.float32),
    )(emb_t)
    return out2.reshape(grid * _TCHUNK, dim)


def _prep_x(x_t):
    seq, batch = x_t.shape
    half = _TCHUNK // 2
    shift_h = half.bit_length() - 1
    tail = seq - 128

    def body(in_ref, out_ref):
        v = in_ref[...]
        blk = v & jnp.int32(-_TCHUNK)
        q = v & jnp.int32(half - 1)
        h = lax.shift_right_logical(v, shift_h) & jnp.int32(1)
        pv = blk + q + q + h
        a = jnp.swapaxes(pv[:128], 0, 1)
        bb = jnp.swapaxes(pv[128:], 0, 1)
        pad = jnp.zeros((batch, 128 - tail), jnp.int32)
        out_ref[...] = jnp.concatenate(
            [a, jnp.concatenate([bb, pad], axis=1)], axis=0)

    return pl.pallas_call(
        body,
        out_shape=jax.ShapeDtypeStruct((2 * batch, 128), jnp.int32),
    )(x_t)


def _sc_pool(xp, emb_lin, batch, seq, dim):
    bpw = batch // _NUM_WORKERS
    tail = seq - 128
    assert 0 < tail <= 128 and tail % 8 == 0

    nvec = dim // _LANES
    mesh = plsc.VectorSubcoreMesh(core_axis_name="c", subcore_axis_name="s")

    @functools.partial(
        pl.kernel,
        mesh=mesh,
        compiler_params=pltpu.CompilerParams(use_tc_tiling_on_sc=False),
        out_type=jax.ShapeDtypeStruct((batch // 2, 2 * dim), jnp.float32),
        scratch_types=[
            pltpu.VMEM((bpw, 128), jnp.int32),
            pltpu.VMEM((bpw, 128), jnp.int32),
            pltpu.VMEM((_NBUF, seq, dim), jnp.float32),
            pltpu.VMEM((bpw // 2, 2 * dim), jnp.float32),
        ] + [pltpu.SemaphoreType.DMA] * _NBUF,
    )
    def pool(xp_hbm, emb_hbm, out_hbm, idx_a, idx_b, rows_v, pooled_v, *sems):
        wid = lax.axis_index("s") * _NUM_CORES + lax.axis_index("c")
        row0 = pl.multiple_of(wid * bpw, 8)
        pltpu.sync_copy(xp_hbm.at[pl.ds(row0, bpw)], idx_a)
        pltpu.sync_copy(xp_hbm.at[pl.ds(batch + row0, bpw)], idx_b)

        def issue(e, b):
            pltpu.async_copy(
                emb_hbm.at[idx_a.at[e]],
                rows_v.at[b].at[pl.ds(0, 128)], sems[b])
            pltpu.async_copy(
                emb_hbm.at[idx_b.at[e, pl.ds(0, tail)]],
                rows_v.at[b].at[pl.ds(128, tail)], sems[b])

        def drain(e, b):
            pltpu.make_async_copy(
                emb_hbm.at[idx_a.at[e]],
                rows_v.at[b].at[pl.ds(0, 128)], sems[b]).wait()
            pltpu.make_async_copy(
                emb_hbm.at[idx_b.at[e, pl.ds(0, tail)]],
                rows_v.at[b].at[pl.ds(128, tail)], sems[b]).wait()

        for b in range(_NBUF):
            issue(b, b)

        def accum(e, b, row, lane0):
            def acc_body(i, accs):
                l0 = i * _UNROLL
                new = list(accs)
                for dl in range(_UNROLL):
                    for v in range(nvec):
                        new[v] = new[v] + rows_v[b, l0 + dl,
                                                 pl.ds(v * _LANES, _LANES)]
                return tuple(new)

            zero = jnp.zeros((_LANES,), jnp.float32)
            accs = lax.fori_loop(0, seq // _UNROLL, acc_body, (zero,) * nvec)
            for l in range(seq - seq % _UNROLL, seq):
                accs = tuple(
                    accs[v] + rows_v[b, l, pl.ds(v * _LANES, _LANES)]
                    for v in range(nvec)
                )
            for v in range(nvec):
                pooled_v[row, pl.ds(lane0 + v * _LANES, _LANES)] = accs[v]

        def do_group(g, carry):
            for b in range(_NBUF):
                e = g * _NBUF + b
                drain(e, b)
                accum(e, b, g * (_NBUF // 2) + b // 2, (b % 2) * dim)

                @pl.when(e + _NBUF < bpw)
                def _():
                    issue(e + _NBUF, b)

            return carry

        ngroups = bpw // _NBUF
        lax.fori_loop(0, ngroups, do_group, 0)
        for t in range(bpw - ngroups * _NBUF):
            e = ngroups * _NBUF + t
            drain(e, t)
            accum(e, t, e // 2, (e % 2) * dim)
        pltpu.sync_copy(pooled_v, out_hbm.at[pl.ds(wid * (bpw // 2), bpw // 2)])

    return pool(xp, emb_lin)


def _mlp_head(pooled2, W1, b1, W2, b2, seq):
    rows, dim2 = pooled2.shape
    dim = dim2 // 2
    hidden = W1.shape[1]

    def body(p_ref, w1_ref, b1_ref, w2_ref, b2_ref, o_ref):
        h2 = p_ref[...] * (1.0 / seq)
        outs = []
        for s in range(2):
            h = h2[:, s * dim:(s + 1) * dim]
            z = jnp.dot(h, w1_ref[...], preferred_element_type=jnp.float32)
            z = jnp.maximum(z + b1_ref[...], 0.0)
            logit = jnp.dot(z, w2_ref[...], preferred_element_type=jnp.float32)
            outs.append(jax.nn.sigmoid(logit + b2_ref[...]))
        o_ref[...] = jnp.concatenate(outs, axis=1)

    out = pl.pallas_call(
        body,
        out_shape=jax.ShapeDtypeStruct((rows, 2), jnp.float32),
    )(pooled2, W1, b1.reshape(1, hidden), W2, b2.reshape(1, 1))
    return out.reshape(2 * rows)


def kernel(x, emb, W1, b1, W2, b2):
    batch, seq = x.shape
    _, dim = emb.shape
    emb_lin = _untile_table(emb.T)
    xp = _prep_x(x.T)
    pooled = _sc_pool(xp, emb_lin, batch, seq, dim)
    return _mlp_head(pooled, W1, b1, W2, b2, seq)


# speedup vs baseline: 1.0196x; 1.0196x over previous
import functools

import jax
import jax.numpy as jnp
from jax import lax
from jax.experimental import pallas as pl
from jax.experimental.pallas import tpu as pltpu
from jax.experimental.pallas import tpu_sc as plsc

_LANES = 16
_NUM_CORES = 2
_NUM_SUBCORES = 16
_NUM_WORKERS = _NUM_CORES * _NUM_SUBCORES
_UNROLL = 20
_NBUF = 4
_TCHUNK = 16384


def _untile_table(emb_t):
    dim, vocab = emb_t.shape
    half = _TCHUNK // 2
    grid = (vocab + _TCHUNK - 1) // _TCHUNK

    def body(in_ref, out_ref):
        b = in_ref[...]
        stacked = jnp.concatenate(
            [b[:, :half], b[:, half:]], axis=0)
        out_ref[...] = jnp.swapaxes(stacked, 0, 1)

    out2 = pl.pallas_call(
        body,
        grid=(grid,),
        in_specs=[pl.BlockSpec((dim, _TCHUNK), lambda c: (0, c))],
        out_specs=pl.BlockSpec((half, 2 * dim), lambda c: (c, 0)),
        out_shape=jax.ShapeDtypeStruct((grid * half, 2 * dim), jnp.float32),
    )(emb_t)
    return out2.reshape(grid * _TCHUNK, dim)


def _prep_x(x_t):
    seq, batch = x_t.shape
    half = _TCHUNK // 2
    shift_h = half.bit_length() - 1
    tail = seq - 128

    def body(in_ref, out_ref):
        v = in_ref[...]
        blk = v & jnp.int32(-_TCHUNK)
        q = v & jnp.int32(half - 1)
        h = lax.shift_right_logical(v, shift_h) & jnp.int32(1)
        pv = blk + q + q + h
        a = jnp.swapaxes(pv[:128], 0, 1)
        bb = jnp.swapaxes(pv[128:], 0, 1)
        pad = jnp.zeros((batch, 128 - tail), jnp.int32)
        out_ref[...] = jnp.concatenate(
            [a, jnp.concatenate([bb, pad], axis=1)], axis=0)

    return pl.pallas_call(
        body,
        out_shape=jax.ShapeDtypeStruct((2 * batch, 128), jnp.int32),
    )(x_t)


def _sc_pool(xp, emb_lin, batch, seq, dim):
    bpw = batch // _NUM_WORKERS
    tail = seq - 128
    assert 0 < tail <= 128 and tail % 8 == 0

    nvec = dim // _LANES
    mesh = plsc.VectorSubcoreMesh(core_axis_name="c", subcore_axis_name="s")

    @functools.partial(
        pl.kernel,
        mesh=mesh,
        compiler_params=pltpu.CompilerParams(use_tc_tiling_on_sc=False),
        out_type=jax.ShapeDtypeStruct((batch // 2, 2 * dim), jnp.float32),
        scratch_types=[
            pltpu.VMEM((bpw, 128), jnp.int32),
            pltpu.VMEM((bpw, 128), jnp.int32),
            pltpu.VMEM((_NBUF, seq, dim), jnp.float32),
            pltpu.VMEM((bpw // 2, 2 * dim), jnp.float32),
        ] + [pltpu.SemaphoreType.DMA] * _NBUF,
    )
    def pool(xp_hbm, emb_hbm, out_hbm, idx_a, idx_b, rows_v, pooled_v, *sems):
        wid = lax.axis_index("s") * _NUM_CORES + lax.axis_index("c")
        row0 = pl.multiple_of(wid * bpw, 8)
        pltpu.sync_copy(xp_hbm.at[pl.ds(row0, bpw)], idx_a)
        pltpu.sync_copy(xp_hbm.at[pl.ds(batch + row0, bpw)], idx_b)

        def issue(e, b):
            pltpu.async_copy(
                emb_hbm.at[idx_a.at[e]],
                rows_v.at[b].at[pl.ds(0, 128)], sems[b])
            pltpu.async_copy(
                emb_hbm.at[idx_b.at[e, pl.ds(0, tail)]],
                rows_v.at[b].at[pl.ds(128, tail)], sems[b])

        def drain(e, b):
            pltpu.make_async_copy(
                emb_hbm.at[idx_a.at[e]],
                rows_v.at[b].at[pl.ds(0, 128)], sems[b]).wait()
            pltpu.make_async_copy(
                emb_hbm.at[idx_b.at[e, pl.ds(0, tail)]],
                rows_v.at[b].at[pl.ds(128, tail)], sems[b]).wait()

        for b in range(_NBUF):
            issue(b, b)

        def accum(e, b, row, lane0):
            def acc_body(i, accs):
                l0 = i * _UNROLL
                new = list(accs)
                for dl in range(_UNROLL):
                    for v in range(nvec):
                        new[v] = new[v] + rows_v[b, l0 + dl,
                                                 pl.ds(v * _LANES, _LANES)]
                return tuple(new)

            zero = jnp.zeros((_LANES,), jnp.float32)
            accs = lax.fori_loop(0, seq // _UNROLL, acc_body, (zero,) * nvec)
            for l in range(seq - seq % _UNROLL, seq):
                accs = tuple(
                    accs[v] + rows_v[b, l, pl.ds(v * _LANES, _LANES)]
                    for v in range(nvec)
                )
            for v in range(nvec):
                pooled_v[row, pl.ds(lane0 + v * _LANES, _LANES)] = accs[v]

        def do_group(g, carry):
            for b in range(_NBUF):
                e = g * _NBUF + b
                drain(e, b)
                accum(e, b, g * (_NBUF // 2) + b // 2, (b % 2) * dim)

                @pl.when(e + _NBUF < bpw)
                def _():
                    issue(e + _NBUF, b)

            return carry

        ngroups = bpw // _NBUF
        lax.fori_loop(0, ngroups, do_group, 0)
        for t in range(bpw - ngroups * _NBUF):
            e = ngroups * _NBUF + t
            drain(e, t)
            accum(e, t, e // 2, (e % 2) * dim)
        pltpu.sync_copy(pooled_v, out_hbm.at[pl.ds(wid * (bpw // 2), bpw // 2)])

    return pool(xp, emb_lin)


def _mlp_head(pooled2, W1, b1, W2, b2, seq):
    rows, dim2 = pooled2.shape
    dim = dim2 // 2
    hidden = W1.shape[1]

    def body(p_ref, w1_ref, b1_ref, w2_ref, b2_ref, o_ref):
        h2 = p_ref[...] * (1.0 / seq)
        outs = []
        for s in range(2):
            h = h2[:, s * dim:(s + 1) * dim]
            z = jnp.dot(h, w1_ref[...], preferred_element_type=jnp.float32)
            z = jnp.maximum(z + b1_ref[...], 0.0)
            logit = jnp.dot(z, w2_ref[...], preferred_element_type=jnp.float32)
            outs.append(jax.nn.sigmoid(logit + b2_ref[...]))
        o_ref[...] = jnp.concatenate(outs, axis=1)

    out = pl.pallas_call(
        body,
        out_shape=jax.ShapeDtypeStruct((rows, 2), jnp.float32),
    )(pooled2, W1, b1.reshape(1, hidden), W2, b2.reshape(1, 1))
    return out.reshape(2 * rows)


def kernel(x, emb, W1, b1, W2, b2):
    batch, seq = x.shape
    _, dim = emb.shape
    emb_lin = _untile_table(emb.T)
    xp = _prep_x(x.T)
    pooled = _sc_pool(xp, emb_lin, batch, seq, dim)
    return _mlp_head(pooled, W1, b1, W2, b2, seq)


# speedup vs baseline: 1.3354x; 1.3098x over previous
import functools

import jax
import jax.numpy as jnp
from jax import lax
from jax.experimental import pallas as pl
from jax.experimental.pallas import tpu as pltpu
from jax.experimental.pallas import tpu_sc as plsc

_LANES = 16
_NUM_CORES = 2
_NUM_SUBCORES = 16
_NUM_WORKERS = _NUM_CORES * _NUM_SUBCORES
_UNROLL = 20
_NBUF = 4
_TCHUNK = 16384


def _untile_table(emb_t):
    dim, vocab = emb_t.shape
    hd = dim // 2
    quarter = _TCHUNK // 4
    grid = (vocab + _TCHUNK - 1) // _TCHUNK

    def body(in_ref, out_ref):
        b = in_ref[...]
        w1 = lax.bitcast_convert_type(
            b[:hd].astype(jnp.bfloat16), jnp.uint16).astype(jnp.int32)
        w2 = lax.bitcast_convert_type(
            b[hd:].astype(jnp.bfloat16), jnp.uint16).astype(jnp.int32)
        packed = (w1 << 16) | w2
        stacked = jnp.concatenate(
            [packed[:, k * quarter:(k + 1) * quarter] for k in range(4)],
            axis=0)
        out_ref[...] = jnp.swapaxes(stacked, 0, 1)

    out2 = pl.pallas_call(
        body,
        grid=(grid,),
        in_specs=[pl.BlockSpec((dim, _TCHUNK), lambda c: (0, c))],
        out_specs=pl.BlockSpec((quarter, 4 * hd), lambda c: (c, 0)),
        out_shape=jax.ShapeDtypeStruct((grid * quarter, 4 * hd), jnp.int32),
    )(emb_t)
    return out2.reshape(grid * _TCHUNK, hd)


def _prep_x(x_t):
    seq, batch = x_t.shape
    quarter = _TCHUNK // 4
    shift_q = quarter.bit_length() - 1
    tail = seq - 128

    def body(in_ref, out_ref):
        v = in_ref[...]
        blk = v & jnp.int32(-_TCHUNK)
        q = v & jnp.int32(quarter - 1)
        h = lax.shift_right_logical(v, shift_q) & jnp.int32(3)
        pv = blk + 4 * q + h
        a = jnp.swapaxes(pv[:128], 0, 1)
        bb = jnp.swapaxes(pv[128:], 0, 1)
        pad = jnp.zeros((batch, 128 - tail), jnp.int32)
        out_ref[...] = jnp.concatenate(
            [a, jnp.concatenate([bb, pad], axis=1)], axis=0)

    return pl.pallas_call(
        body,
        out_shape=jax.ShapeDtypeStruct((2 * batch, 128), jnp.int32),
    )(x_t)


def _sc_pool(xp, emb_lin, batch, seq, dim):
    bpw = batch // _NUM_WORKERS
    tail = seq - 128
    assert 0 < tail <= 128 and tail % 8 == 0

    nvec = dim // _LANES
    mesh = plsc.VectorSubcoreMesh(core_axis_name="c", subcore_axis_name="s")

    @functools.partial(
        pl.kernel,
        mesh=mesh,
        compiler_params=pltpu.CompilerParams(
            use_tc_tiling_on_sc=False, needs_layout_passes=False),
        out_type=jax.ShapeDtypeStruct((batch // 2, 2 * dim), jnp.float32),
        scratch_types=[
            pltpu.VMEM((bpw, 128), jnp.int32),
            pltpu.VMEM((bpw, 128), jnp.int32),
            pltpu.VMEM((_NBUF, seq, dim // 2), jnp.int32),
            pltpu.VMEM((bpw // 2, 2 * dim), jnp.float32),
        ] + [pltpu.SemaphoreType.DMA] * _NBUF,
    )
    def pool(xp_hbm, emb_hbm, out_hbm, idx_a, idx_b, rows_v, pooled_v, *sems):
        wid = lax.axis_index("s") * _NUM_CORES + lax.axis_index("c")
        row0 = pl.multiple_of(wid * bpw, 8)
        pltpu.sync_copy(xp_hbm.at[pl.ds(row0, bpw)], idx_a)
        pltpu.sync_copy(xp_hbm.at[pl.ds(batch + row0, bpw)], idx_b)

        def issue(e, b):
            pltpu.async_copy(
                emb_hbm.at[idx_a.at[e]],
                rows_v.at[b].at[pl.ds(0, 128)], sems[b])
            pltpu.async_copy(
                emb_hbm.at[idx_b.at[e, pl.ds(0, tail)]],
                rows_v.at[b].at[pl.ds(128, tail)], sems[b])

        def drain(e, b):
            pltpu.make_async_copy(
                emb_hbm.at[idx_a.at[e]],
                rows_v.at[b].at[pl.ds(0, 128)], sems[b]).wait()
            pltpu.make_async_copy(
                emb_hbm.at[idx_b.at[e, pl.ds(0, tail)]],
                rows_v.at[b].at[pl.ds(128, tail)], sems[b]).wait()

        for b in range(_NBUF):
            issue(b, b)

        hi_mask = jnp.int32(-65536)

        def unpacked_terms(b, l):
            terms = []
            for w in range(nvec // 2):
                v = rows_v[b, l, pl.ds(w * _LANES, _LANES)]
                terms.append((w, plsc.bitcast(v & hi_mask, jnp.float32)))
                terms.append((nvec // 2 + w,
                              plsc.bitcast(v << 16, jnp.float32)))
            return terms

        def accum(e, b, row, lane0):
            def acc_body(i, accs):
                l0 = i * _UNROLL
                new = list(accs)
                for dl in range(_UNROLL):
                    for v, t in unpacked_terms(b, l0 + dl):
                        new[v] = new[v] + t
                return tuple(new)

            zero = jnp.zeros((_LANES,), jnp.float32)
            accs = lax.fori_loop(0, seq // _UNROLL, acc_body, (zero,) * nvec)
            for l in range(seq - seq % _UNROLL, seq):
                new = list(accs)
                for v, t in unpacked_terms(b, l):
                    new[v] = new[v] + t
                accs = tuple(new)
            for v in range(nvec):
                pooled_v[row, pl.ds(lane0 + v * _LANES, _LANES)] = accs[v]

        def do_group(g, carry):
            for b in range(_NBUF):
                e = g * _NBUF + b
                drain(e, b)
                accum(e, b, g * (_NBUF // 2) + b // 2, (b % 2) * dim)

                @pl.when(e + _NBUF < bpw)
                def _():
                    issue(e + _NBUF, b)

            return carry

        ngroups = bpw // _NBUF
        lax.fori_loop(0, ngroups, do_group, 0)
        for t in range(bpw - ngroups * _NBUF):
            e = ngroups * _NBUF + t
            drain(e, t)
            accum(e, t, e // 2, (e % 2) * dim)
        pltpu.sync_copy(pooled_v, out_hbm.at[pl.ds(wid * (bpw // 2), bpw // 2)])

    return pool(xp, emb_lin)


def _mlp_head(pooled2, W1, b1, W2, b2, seq):
    rows, dim2 = pooled2.shape
    dim = dim2 // 2
    hidden = W1.shape[1]

    def body(p_ref, w1_ref, b1_ref, w2_ref, b2_ref, o_ref):
        h2 = p_ref[...] * (1.0 / seq)
        outs = []
        for s in range(2):
            h = h2[:, s * dim:(s + 1) * dim]
            z = jnp.dot(h, w1_ref[...], preferred_element_type=jnp.float32)
            z = jnp.maximum(z + b1_ref[...], 0.0)
            logit = jnp.dot(z, w2_ref[...], preferred_element_type=jnp.float32)
            outs.append(jax.nn.sigmoid(logit + b2_ref[...]))
        o_ref[...] = jnp.concatenate(outs, axis=1)

    out = pl.pallas_call(
        body,
        out_shape=jax.ShapeDtypeStruct((rows, 2), jnp.float32),
    )(pooled2, W1, b1.reshape(1, hidden), W2, b2.reshape(1, 1))
    return out.reshape(2 * rows)


def kernel(x, emb, W1, b1, W2, b2):
    batch, seq = x.shape
    _, dim = emb.shape
    emb_lin = _untile_table(emb.T)
    xp = _prep_x(x.T)
    pooled = _sc_pool(xp, emb_lin, batch, seq, dim)
    return _mlp_head(pooled, W1, b1, W2, b2, seq)


# speedup vs baseline: 1.3583x; 1.0172x over previous
import functools

import jax
import jax.numpy as jnp
from jax import lax
from jax.experimental import pallas as pl
from jax.experimental.pallas import tpu as pltpu
from jax.experimental.pallas import tpu_sc as plsc

_LANES = 16
_NUM_CORES = 2
_NUM_SUBCORES = 16
_NUM_WORKERS = _NUM_CORES * _NUM_SUBCORES
_UNROLL = 20
_NBUF = 8
_TCHUNK = 16384


def _untile_table(emb_t):
    dim, vocab = emb_t.shape
    hd = dim // 2
    quarter = _TCHUNK // 4
    grid = (vocab + _TCHUNK - 1) // _TCHUNK

    def body(in_ref, out_ref):
        b = in_ref[...]
        w1 = lax.bitcast_convert_type(
            b[:hd].astype(jnp.bfloat16), jnp.uint16).astype(jnp.int32)
        w2 = lax.bitcast_convert_type(
            b[hd:].astype(jnp.bfloat16), jnp.uint16).astype(jnp.int32)
        packed = (w1 << 16) | w2
        stacked = jnp.concatenate(
            [packed[:, k * quarter:(k + 1) * quarter] for k in range(4)],
            axis=0)
        out_ref[...] = jnp.swapaxes(stacked, 0, 1)

    out2 = pl.pallas_call(
        body,
        grid=(grid,),
        in_specs=[pl.BlockSpec((dim, _TCHUNK), lambda c: (0, c))],
        out_specs=pl.BlockSpec((quarter, 4 * hd), lambda c: (c, 0)),
        out_shape=jax.ShapeDtypeStruct((grid * quarter, 4 * hd), jnp.int32),
    )(emb_t)
    return out2.reshape(grid * _TCHUNK, hd)


def _prep_x(x_t):
    seq, batch = x_t.shape
    quarter = _TCHUNK // 4
    shift_q = quarter.bit_length() - 1
    tail = seq - 128

    def body(in_ref, out_ref):
        v = in_ref[...]
        blk = v & jnp.int32(-_TCHUNK)
        q = v & jnp.int32(quarter - 1)
        h = lax.shift_right_logical(v, shift_q) & jnp.int32(3)
        pv = blk + 4 * q + h
        a = jnp.swapaxes(pv[:128], 0, 1)
        bb = jnp.swapaxes(pv[128:], 0, 1)
        pad = jnp.zeros((batch, 128 - tail), jnp.int32)
        out_ref[...] = jnp.concatenate(
            [a, jnp.concatenate([bb, pad], axis=1)], axis=0)

    return pl.pallas_call(
        body,
        out_shape=jax.ShapeDtypeStruct((2 * batch, 128), jnp.int32),
    )(x_t)


def _sc_pool(xp, emb_lin, batch, seq, dim):
    bpw = batch // _NUM_WORKERS
    tail = seq - 128
    assert 0 < tail <= 128 and tail % 8 == 0

    nvec = dim // _LANES
    mesh = plsc.VectorSubcoreMesh(core_axis_name="c", subcore_axis_name="s")

    @functools.partial(
        pl.kernel,
        mesh=mesh,
        compiler_params=pltpu.CompilerParams(
            use_tc_tiling_on_sc=False, needs_layout_passes=False),
        out_type=jax.ShapeDtypeStruct((batch // 2, 2 * dim), jnp.float32),
        scratch_types=[
            pltpu.VMEM((bpw, 128), jnp.int32),
            pltpu.VMEM((bpw, 128), jnp.int32),
            pltpu.VMEM((_NBUF, seq, dim // 2), jnp.int32),
            pltpu.VMEM((bpw // 2, 2 * dim), jnp.float32),
        ] + [pltpu.SemaphoreType.DMA] * _NBUF,
    )
    def pool(xp_hbm, emb_hbm, out_hbm, idx_a, idx_b, rows_v, pooled_v, *sems):
        wid = lax.axis_index("s") * _NUM_CORES + lax.axis_index("c")
        row0 = pl.multiple_of(wid * bpw, 8)
        pltpu.sync_copy(xp_hbm.at[pl.ds(row0, bpw)], idx_a)
        pltpu.sync_copy(xp_hbm.at[pl.ds(batch + row0, bpw)], idx_b)

        def issue(e, b):
            pltpu.async_copy(
                emb_hbm.at[idx_a.at[e]],
                rows_v.at[b].at[pl.ds(0, 128)], sems[b])
            pltpu.async_copy(
                emb_hbm.at[idx_b.at[e, pl.ds(0, tail)]],
                rows_v.at[b].at[pl.ds(128, tail)], sems[b])

        def drain(e, b):
            pltpu.make_async_copy(
                emb_hbm.at[idx_a.at[e]],
                rows_v.at[b].at[pl.ds(0, 128)], sems[b]).wait()
            pltpu.make_async_copy(
                emb_hbm.at[idx_b.at[e, pl.ds(0, tail)]],
                rows_v.at[b].at[pl.ds(128, tail)], sems[b]).wait()

        for b in range(_NBUF):
            issue(b, b)

        hi_mask = jnp.int32(-65536)

        def unpacked_terms(b, l):
            terms = []
            for w in range(nvec // 2):
                v = rows_v[b, l, pl.ds(w * _LANES, _LANES)]
                terms.append((w, plsc.bitcast(v & hi_mask, jnp.float32)))
                terms.append((nvec // 2 + w,
                              plsc.bitcast(v << 16, jnp.float32)))
            return terms

        def accum(e, b, row, lane0):
            def acc_body(i, accs):
                l0 = i * _UNROLL
                new = list(accs)
                for dl in range(_UNROLL):
                    for v, t in unpacked_terms(b, l0 + dl):
                        new[v] = new[v] + t
                return tuple(new)

            zero = jnp.zeros((_LANES,), jnp.float32)
            accs = lax.fori_loop(0, seq // _UNROLL, acc_body, (zero,) * nvec)
            for l in range(seq - seq % _UNROLL, seq):
                new = list(accs)
                for v, t in unpacked_terms(b, l):
                    new[v] = new[v] + t
                accs = tuple(new)
            for v in range(nvec):
                pooled_v[row, pl.ds(lane0 + v * _LANES, _LANES)] = accs[v]

        def do_group(g, carry):
            for b in range(_NBUF):
                e = g * _NBUF + b
                drain(e, b)
                accum(e, b, g * (_NBUF // 2) + b // 2, (b % 2) * dim)

                @pl.when(e + _NBUF < bpw)
                def _():
                    issue(e + _NBUF, b)

            return carry

        ngroups = bpw // _NBUF
        lax.fori_loop(0, ngroups, do_group, 0)
        for t in range(bpw - ngroups * _NBUF):
            e = ngroups * _NBUF + t
            drain(e, t)
            accum(e, t, e // 2, (e % 2) * dim)
        pltpu.sync_copy(pooled_v, out_hbm.at[pl.ds(wid * (bpw // 2), bpw // 2)])

    return pool(xp, emb_lin)


def _mlp_head(pooled2, W1, b1, W2, b2, seq):
    rows, dim2 = pooled2.shape
    dim = dim2 // 2
    hidden = W1.shape[1]

    def body(p_ref, w1_ref, b1_ref, w2_ref, b2_ref, o_ref):
        h2 = p_ref[...] * (1.0 / seq)
        outs = []
        for s in range(2):
            h = h2[:, s * dim:(s + 1) * dim]
            z = jnp.dot(h, w1_ref[...], preferred_element_type=jnp.float32)
            z = jnp.maximum(z + b1_ref[...], 0.0)
            logit = jnp.dot(z, w2_ref[...], preferred_element_type=jnp.float32)
            outs.append(jax.nn.sigmoid(logit + b2_ref[...]))
        o_ref[...] = jnp.concatenate(outs, axis=1)

    out = pl.pallas_call(
        body,
        out_shape=jax.ShapeDtypeStruct((rows, 2), jnp.float32),
    )(pooled2, W1, b1.reshape(1, hidden), W2, b2.reshape(1, 1))
    return out.reshape(2 * rows)


def kernel(x, emb, W1, b1, W2, b2):
    batch, seq = x.shape
    _, dim = emb.shape
    emb_lin = _untile_table(emb.T)
    xp = _prep_x(x.T)
    pooled = _sc_pool(xp, emb_lin, batch, seq, dim)
    return _mlp_head(pooled, W1, b1, W2, b2, seq)
